# baseline (speedup 1.0000x reference)
import functools

import jax
import jax.numpy as jnp
from jax import lax
from jax.experimental import pallas as pl
from jax.experimental.pallas import tpu as pltpu
from jax.experimental.pallas import tpu_sc as plsc

VOCAB = 1000
BATCH = 4096
HIST = 20
PRIOR = 1.0


def _table_body(n_ref, p_ref):
    n = n_ref[...] + PRIOR
    s = jnp.sum(n, axis=1, keepdims=True)
    p = jnp.log(n / s)
    p_ref[...] = jnp.maximum(p, -1.0e6)


def _compute_table(N):
    return pl.pallas_call(
        _table_body,
        out_shape=jax.ShapeDtypeStruct((VOCAB, VOCAB), jnp.float32),
        in_specs=[pl.BlockSpec(memory_space=pltpu.VMEM)],
        out_specs=pl.BlockSpec(memory_space=pltpu.VMEM),
    )(N)


_B = BATCH * HIST


def _make_gather(V, D, B):
    info = plsc.get_sparse_core_info()
    NC, NS = info.num_cores, info.num_subcores
    NW = NC * NS
    assert B % NW == 0
    per_w = B // NW
    CH = 40
    assert per_w % (2 * CH) == 0
    n2 = per_w // (2 * CH)
    SPC = CH // HIST
    mesh = plsc.VectorSubcoreMesh(core_axis_name="c", subcore_axis_name="s")

    @functools.partial(
        pl.kernel,
        mesh=mesh,
        out_type=jax.ShapeDtypeStruct((BATCH, HIST, D), jnp.float32),
        scratch_types=[
            pltpu.VMEM((per_w,), jnp.int32),
            pltpu.VMEM((CH, D), jnp.float32),
            pltpu.VMEM((CH, D), jnp.float32),
            pltpu.SemaphoreType.DMA,
            pltpu.SemaphoreType.DMA,
            pltpu.SemaphoreType.DMA,
            pltpu.SemaphoreType.DMA,
        ],
        compiler_params=pltpu.CompilerParams(use_tc_tiling_on_sc=False),
    )
    def gather(table_hbm, idx_hbm, out3d_hbm, idx_v, buf0, buf1,
               sg0, sg1, so0, so1):
        wid = lax.axis_index("s") * NC + lax.axis_index("c")
        base = wid * per_w
        sbase = wid * (per_w // HIST)
        pltpu.sync_copy(idx_hbm.at[pl.ds(base, per_w)], idx_v)

        def wait_bytes(sem):
            pltpu.make_async_copy(table_hbm.at[pl.ds(0, CH)],
                                  buf0, sem).wait()

        def copy_out(buf, s0, sem):
            for j in range(SPC):
                pltpu.async_copy(buf.at[pl.ds(j * HIST, HIST)],
                                 out3d_hbm.at[s0 + j], sem)

        pltpu.async_copy(table_hbm.at[idx_v.at[pl.ds(0, CH)]], buf0, sg0)

        def body(k, carry):
            c0 = 2 * k * CH
            c1 = c0 + CH

            @pl.when(k > 0)
            def _():
                wait_bytes(so1)
            pltpu.async_copy(
                table_hbm.at[idx_v.at[pl.ds(c1, CH)]], buf1, sg1)
            wait_bytes(sg0)
            copy_out(buf0, sbase + 2 * k * SPC, so0)
            wait_bytes(sg1)
            copy_out(buf1, sbase + (2 * k + 1) * SPC, so1)
            wait_bytes(so0)

            @pl.when(k + 1 < n2)
            def _():
                pltpu.async_copy(
                    table_hbm.at[idx_v.at[pl.ds(c1 + CH, CH)]], buf0, sg0)
            return carry

        lax.fori_loop(0, n2, body, 0)
        wait_bytes(so1)

    return gather


_gather = _make_gather(VOCAB, VOCAB, _B)


def kernel(N, idx):
    p = _compute_table(N.astype(jnp.float32))
    flat = idx.reshape(-1).astype(jnp.int32)
    return _gather(p, flat)


# speedup vs baseline: 1.3808x; 1.3808x over previous
import functools

import jax
import jax.numpy as jnp
from jax import lax
from jax.experimental import pallas as pl
from jax.experimental.pallas import tpu as pltpu
from jax.experimental.pallas import tpu_sc as plsc

VOCAB = 1000
BATCH = 4096
HIST = 20
PRIOR = 1.0
_DPAD = 1024


def _table_body(n_ref, p_ref):
    n = n_ref[...] + PRIOR
    s = jnp.sum(n, axis=1, keepdims=True)
    p = jnp.log(n / s)
    p = jnp.maximum(p, -1.0e6)
    p_ref[...] = jnp.pad(p, ((0, 0), (0, _DPAD - VOCAB)))


def _compute_table(N):
    return pl.pallas_call(
        _table_body,
        out_shape=jax.ShapeDtypeStruct((VOCAB, _DPAD), jnp.float32),
        in_specs=[pl.BlockSpec(memory_space=pltpu.VMEM)],
        out_specs=pl.BlockSpec(memory_space=pltpu.VMEM),
    )(N)


def _make_gather():
    info = plsc.get_sparse_core_info()
    NC, NS = info.num_cores, info.num_subcores
    NW = NC * NS
    assert BATCH % NW == 0
    spw = BATCH // NW
    n2 = spw // 2
    L = info.num_lanes
    nfull = VOCAB // L
    tail = VOCAB - L
    mesh = plsc.VectorSubcoreMesh(core_axis_name="c", subcore_axis_name="s")

    @functools.partial(
        pl.kernel,
        mesh=mesh,
        out_type=jax.ShapeDtypeStruct((BATCH, HIST, VOCAB), jnp.float32),
        scratch_types=[
            pltpu.VMEM((NW * 4, HIST), jnp.int32),
            pltpu.VMEM((HIST, _DPAD), jnp.float32),
            pltpu.VMEM((HIST, _DPAD), jnp.float32),
            pltpu.VMEM((HIST, VOCAB), jnp.float32),
            pltpu.VMEM((HIST, VOCAB), jnp.float32),
            pltpu.SemaphoreType.DMA,
            pltpu.SemaphoreType.DMA,
            pltpu.SemaphoreType.DMA,
            pltpu.SemaphoreType.DMA,
        ],
    )
    def gather(table_hbm, idx_hbm, out_hbm, idx_v, ga0, ga1, nb0, nb1,
               sg0, sg1, so0, so1):
        wid = lax.axis_index("s") * NC + lax.axis_index("c")
        sbase = wid * spw
        pltpu.sync_copy(idx_hbm.at[pl.ds(sbase, spw)], idx_v)

        def wait_gather(sem, ga):
            pltpu.make_async_copy(table_hbm.at[idx_v.at[0]], ga,
                                  sem).wait()

        def wait_out(sem, nb):
            pltpu.make_async_copy(nb, out_hbm.at[sbase], sem).wait()

        def narrow(ga, nb):
            def row(r, carry):
                for c in range(nfull):
                    nb[r, pl.ds(c * L, L)] = ga[r, pl.ds(c * L, L)]
                nb[r, pl.ds(tail, L)] = ga[r, pl.ds(tail, L)]
                return carry
            lax.fori_loop(0, HIST, row, 0)

        pltpu.async_copy(table_hbm.at[idx_v.at[0]], ga0, sg0)

        def body(k, carry):
            s0 = 2 * k
            s1 = s0 + 1
            pltpu.async_copy(table_hbm.at[idx_v.at[s1]], ga1, sg1)
            wait_gather(sg0, ga0)

            @pl.when(k > 0)
            def _():
                wait_out(so0, nb0)
            narrow(ga0, nb0)
            pltpu.async_copy(nb0, out_hbm.at[sbase + s0], so0)

            @pl.when(k + 1 < n2)
            def _():
                pltpu.async_copy(table_hbm.at[idx_v.at[s0 + 2]], ga0, sg0)
            wait_gather(sg1, ga1)

            @pl.when(k > 0)
            def _():
                wait_out(so1, nb1)
            narrow(ga1, nb1)
            pltpu.async_copy(nb1, out_hbm.at[sbase + s1], so1)
            return carry

        lax.fori_loop(0, n2, body, 0)
        wait_out(so0, nb0)
        wait_out(so1, nb1)

    return gather


_gather = _make_gather()


def kernel(N, idx):
    p = _compute_table(N.astype(jnp.float32))
    return _gather(p, idx.astype(jnp.int32))


# speedup vs baseline: 1.6272x; 1.1784x over previous
import functools

import jax
import jax.numpy as jnp
from jax import lax
from jax.experimental import pallas as pl
from jax.experimental.pallas import tpu as pltpu
from jax.experimental.pallas import tpu_sc as plsc

VOCAB = 1000
BATCH = 4096
HIST = 20
PRIOR = 1.0
_DPAD = 1024


def _table_body(n_ref, p_ref):
    n = n_ref[...] + PRIOR
    s = jnp.sum(n, axis=1, keepdims=True)
    p = jnp.log(n / s)
    p = jnp.maximum(p, -1.0e6)
    p_ref[...] = jnp.pad(p, ((0, 0), (0, _DPAD - VOCAB)))


def _compute_table(N):
    return pl.pallas_call(
        _table_body,
        out_shape=jax.ShapeDtypeStruct((VOCAB, _DPAD), jnp.float32),
        in_specs=[pl.BlockSpec(memory_space=pltpu.VMEM)],
        out_specs=pl.BlockSpec(memory_space=pltpu.VMEM),
    )(N)


def _make_gather():
    info = plsc.get_sparse_core_info()
    NC, NS = info.num_cores, info.num_subcores
    NW = NC * NS
    assert BATCH % NW == 0
    spw = BATCH // NW
    n2 = spw // 2
    L = info.num_lanes
    nfull = VOCAB // L
    tail = VOCAB - L
    mesh = plsc.VectorSubcoreMesh(core_axis_name="c", subcore_axis_name="s")

    @functools.partial(
        pl.kernel,
        mesh=mesh,
        out_type=jax.ShapeDtypeStruct((BATCH, HIST, VOCAB), jnp.float32),
        scratch_types=[
            pltpu.VMEM((NW * 4, HIST), jnp.int32),
            pltpu.VMEM((HIST, _DPAD), jnp.float32),
            pltpu.VMEM((HIST, _DPAD), jnp.float32),
            pltpu.VMEM((HIST, VOCAB), jnp.float32),
            pltpu.VMEM((HIST, VOCAB), jnp.float32),
            pltpu.SemaphoreType.DMA,
            pltpu.SemaphoreType.DMA,
            pltpu.SemaphoreType.DMA,
            pltpu.SemaphoreType.DMA,
        ],
    )
    def gather(table_hbm, idx_hbm, out_hbm, idx_v, ga0, ga1, nb0, nb1,
               sg0, sg1, so0, so1):
        wid = lax.axis_index("s") * NC + lax.axis_index("c")
        sbase = wid * spw
        pltpu.sync_copy(idx_hbm.at[pl.ds(sbase, spw)], idx_v)

        def wait_gather(sem, ga):
            pltpu.make_async_copy(table_hbm.at[idx_v.at[0]], ga,
                                  sem).wait()

        def wait_out(sem, nb):
            pltpu.make_async_copy(nb, out_hbm.at[sbase], sem).wait()

        def narrow(ga, nb):
            def row(r, carry):
                for c in range(nfull):
                    nb[r, pl.ds(c * L, L)] = ga[r, pl.ds(c * L, L)]
                nb[r, pl.ds(tail, L)] = ga[r, pl.ds(tail, L)]
                return carry
            lax.fori_loop(0, HIST, row, 0)

        pltpu.async_copy(table_hbm.at[idx_v.at[0]], ga0, sg0)

        def body(k, carry):
            s0 = 2 * k
            s1 = s0 + 1
            pltpu.async_copy(table_hbm.at[idx_v.at[s1]], ga1, sg1)
            wait_gather(sg0, ga0)

            @pl.when(k > 0)
            def _():
                wait_out(so0, nb0)
            narrow(ga0, nb0)
            pltpu.async_copy(nb0, out_hbm.at[sbase + s0], so0)

            @pl.when(k + 1 < n2)
            def _():
                pltpu.async_copy(table_hbm.at[idx_v.at[s0 + 2]], ga0, sg0)
            wait_gather(sg1, ga1)

            @pl.when(k > 0)
            def _():
                wait_out(so1, nb1)
            narrow(ga1, nb1)
            pltpu.async_copy(nb1, out_hbm.at[sbase + s1], so1)
            return carry

        lax.fori_loop(0, n2, body, 0)
        wait_out(so0, nb0)
        wait_out(so1, nb1)

    return gather


_gather = _make_gather()


def kernel(N, idx):
    p = _compute_table(N.astype(jnp.float32))
    out = _gather(p, idx.astype(jnp.int32))
    return jax.lax.optimization_barrier(out)
